# baseline (speedup 1.0000x reference)
import functools

import jax
import jax.numpy as jnp
from jax import lax
from jax.experimental import pallas as pl
from jax.experimental.pallas import tpu as pltpu
from jax.experimental.pallas import tpu_sc as plsc

P = 7
LANES = 16


def _roi_pool_sc(fmap2, meta, n_boxes, C, H, W, n_blk, bpb):
    half_c = C // 2
    n_grp = half_c // LANES
    out_row = half_c * P * P

    mesh = plsc.VectorSubcoreMesh(core_axis_name="c", subcore_axis_name="s")

    @functools.partial(
        pl.kernel,
        out_type=jax.ShapeDtypeStruct((n_boxes, 2, out_row), jnp.float32),
        mesh=mesh,
        scratch_types=[
            pltpu.VMEM((H * W * half_c,), jnp.float32),
            pltpu.VMEM((bpb * 2 * LANES,), jnp.int32),
            pltpu.VMEM((2, out_row), jnp.float32),
            pltpu.SemaphoreType.DMA,
            pltpu.SemaphoreType.DMA,
        ],
    )
    def k(fmap_hbm, meta_hbm, out_hbm, table_v, meta_v, out_v, sem0, sem1):
        half = lax.axis_index("c")
        blk = lax.axis_index("s")
        pltpu.sync_copy(fmap_hbm.at[half], table_v)
        pltpu.sync_copy(meta_hbm.at[blk], meta_v)
        neg = jnp.full((LANES,), -jnp.inf, dtype=jnp.float32)
        zero = jnp.zeros((LANES,), dtype=jnp.float32)

        nbox = jnp.minimum(bpb, n_boxes - blk * bpb)

        def box_body(i, carry):
            par = i % 2
            par0 = par == 0
            n = blk * bpb + i
            @pl.when((i >= 2) & par0)
            def _():
                pltpu.make_async_copy(
                    out_v.at[0], out_hbm.at[n, half], sem0
                ).wait()

            @pl.when((i >= 2) & jnp.logical_not(par0))
            def _():
                pltpu.make_async_copy(
                    out_v.at[1], out_hbm.at[n, half], sem1
                ).wait()

            vh = meta_v[pl.ds(i * 2 * LANES, LANES)]
            vw = meta_v[pl.ds(i * 2 * LANES + LANES, LANES)]
            for ph in range(P):
                hs = vh[ph]
                he = vh[P + ph]
                for pw in range(P):
                    ws = vw[pw]
                    we = vw[P + pw]
                    binidx = ph * P + pw

                    def h_body(h, accs, ws=ws, we=we):
                        base = h * W

                        def w_body(w, accs):
                            pix = (base + w) * half_c
                            return tuple(
                                jnp.maximum(
                                    accs[g],
                                    table_v[pl.ds(pix + g * LANES, LANES)],
                                )
                                for g in range(n_grp)
                            )

                        return lax.fori_loop(ws, we, w_body, accs)

                    accs = lax.fori_loop(hs, he, h_body, (neg,) * n_grp)
                    valid = (hs < he) & (ws < we)
                    for g in range(n_grp):
                        res = jnp.where(valid, accs[g], zero)
                        off = binidx * half_c + g * LANES
                        out_v[par, pl.ds(off, LANES)] = res
            @pl.when(par0)
            def _():
                pltpu.async_copy(out_v.at[0], out_hbm.at[n, half], sem0)

            @pl.when(jnp.logical_not(par0))
            def _():
                pltpu.async_copy(out_v.at[1], out_hbm.at[n, half], sem1)

            return carry

        lax.fori_loop(0, nbox, box_body, 0)
        last = blk * bpb
        pltpu.make_async_copy(
            out_v.at[0], out_hbm.at[last, half], sem0
        ).wait()
        pltpu.make_async_copy(
            out_v.at[1], out_hbm.at[last, half], sem1
        ).wait()

    return k(fmap2, meta)


def kernel(feature, boxes, image_size):
    C, H, W = feature.shape[1], feature.shape[2], feature.shape[3]
    N = boxes.shape[0]
    half_c = C // 2

    ih = image_size[0].astype(jnp.float32)
    iw = image_size[1].astype(jnp.float32)
    scale = jnp.minimum(jnp.float32(H), jnp.float32(W)) / jnp.minimum(ih, iw)

    r = jnp.round(boxes * scale).astype(jnp.int32)
    rsw, rsh, rew, reh = r[:, 0], r[:, 1], r[:, 2], r[:, 3]
    roi_w = jnp.maximum(rew - rsw + 1, 1)[:, None]
    roi_h = jnp.maximum(reh - rsh + 1, 1)[:, None]
    bins = jnp.arange(P)[None, :]
    hs = jnp.clip((bins * roi_h) // P + rsh[:, None], 0, H)
    he = jnp.clip(-((-(bins + 1) * roi_h) // P) + rsh[:, None], 0, H)
    ws = jnp.clip((bins * roi_w) // P + rsw[:, None], 0, W)
    we = jnp.clip(-((-(bins + 1) * roi_w) // P) + rsw[:, None], 0, W)
    pad2 = jnp.zeros((N, LANES - 2 * P), jnp.int32)
    vh = jnp.concatenate([hs, he, pad2], axis=1).astype(jnp.int32)
    vw = jnp.concatenate([ws, we, pad2], axis=1).astype(jnp.int32)
    meta = jnp.stack([vh, vw], axis=1)

    n_blk = 16
    bpb = -(-N // (n_blk * 4)) * 4
    meta = jnp.pad(meta, ((0, n_blk * bpb - N), (0, 0), (0, 0)))
    meta = meta.reshape(n_blk, bpb * 2 * LANES)

    fmap2 = (
        feature[0]
        .transpose(1, 2, 0)
        .reshape(H * W, 2, half_c)
        .transpose(1, 0, 2)
        .reshape(2, H * W * half_c)
    )

    out = _roi_pool_sc(fmap2, meta, N, C, H, W, n_blk, bpb)
    out = out.reshape(N, 2, P * P, half_c).transpose(0, 1, 3, 2)
    return out.reshape(N, C, P, P)


# speedup vs baseline: 1.2858x; 1.2858x over previous
import functools

import jax
import jax.numpy as jnp
from jax import lax
from jax.experimental import pallas as pl
from jax.experimental.pallas import tpu as pltpu
from jax.experimental.pallas import tpu_sc as plsc

P = 7
LANES = 16


def _roi_pool_sc(fmap2, meta, n_boxes, C, H, W, n_blk, bpb):
    half_c = C // 2
    n_grp = half_c // LANES
    out_row = half_c * P * P

    mesh = plsc.VectorSubcoreMesh(core_axis_name="c", subcore_axis_name="s")

    @functools.partial(
        pl.kernel,
        out_type=jax.ShapeDtypeStruct((n_boxes, 2, out_row), jnp.float32),
        mesh=mesh,
        scratch_types=[
            pltpu.VMEM((H * W * half_c,), jnp.float32),
            pltpu.VMEM((bpb * 2 * LANES,), jnp.int32),
            pltpu.VMEM((2, out_row), jnp.float32),
            pltpu.SemaphoreType.DMA,
            pltpu.SemaphoreType.DMA,
        ],
    )
    def k(fmap_hbm, meta_hbm, out_hbm, table_v, meta_v, out_v, sem0, sem1):
        half = lax.axis_index("c")
        blk = lax.axis_index("s")
        pltpu.sync_copy(fmap_hbm.at[half], table_v)
        pltpu.sync_copy(meta_hbm.at[blk], meta_v)
        neg = jnp.full((LANES,), -jnp.inf, dtype=jnp.float32)
        zero = jnp.zeros((LANES,), dtype=jnp.float32)
        iota = lax.iota(jnp.int32, LANES)
        rot = (iota + 1) % LANES

        nbox = jnp.minimum(bpb, n_boxes - blk * bpb)

        def box_body(i, carry):
            par = i % 2
            par0 = par == 0
            n = blk * bpb + i
            @pl.when((i >= 2) & par0)
            def _():
                pltpu.make_async_copy(
                    out_v.at[0], out_hbm.at[n, half], sem0
                ).wait()

            @pl.when((i >= 2) & jnp.logical_not(par0))
            def _():
                pltpu.make_async_copy(
                    out_v.at[1], out_hbm.at[n, half], sem1
                ).wait()

            vh = meta_v[pl.ds(i * 2 * LANES, LANES)]
            vw = meta_v[pl.ds(i * 2 * LANES + LANES, LANES)]

            def ph_body(ph, vhc):
                hs = vhc[0]
                he = vhc[P]

                def pw_body(pw, vwc, hs=hs, he=he, ph=ph):
                    ws = vwc[0]
                    we = vwc[P]

                    def h_body(h, accs, ws=ws, we=we):
                        base = h * W

                        def w_body(w, accs):
                            pix = (base + w) * half_c
                            return tuple(
                                jnp.maximum(
                                    accs[g],
                                    table_v[pl.ds(pix + g * LANES, LANES)],
                                )
                                for g in range(n_grp)
                            )

                        return lax.fori_loop(ws, we, w_body, accs)

                    accs = lax.fori_loop(hs, he, h_body, (neg,) * n_grp)
                    valid = (hs < he) & (ws < we)
                    off0 = (ph * P + pw) * half_c
                    for g in range(n_grp):
                        res = jnp.where(valid, accs[g], zero)
                        out_v[par, pl.ds(off0 + g * LANES, LANES)] = res
                    return vwc.at[rot].get(mode="promise_in_bounds")

                lax.fori_loop(0, P, pw_body, vw)
                return vhc.at[rot].get(mode="promise_in_bounds")

            lax.fori_loop(0, P, ph_body, vh)
            @pl.when(par0)
            def _():
                pltpu.async_copy(out_v.at[0], out_hbm.at[n, half], sem0)

            @pl.when(jnp.logical_not(par0))
            def _():
                pltpu.async_copy(out_v.at[1], out_hbm.at[n, half], sem1)

            return carry

        lax.fori_loop(0, nbox, box_body, 0)
        last = blk * bpb
        pltpu.make_async_copy(
            out_v.at[0], out_hbm.at[last, half], sem0
        ).wait()
        pltpu.make_async_copy(
            out_v.at[1], out_hbm.at[last, half], sem1
        ).wait()

    return k(fmap2, meta)


def kernel(feature, boxes, image_size):
    C, H, W = feature.shape[1], feature.shape[2], feature.shape[3]
    N = boxes.shape[0]
    half_c = C // 2

    ih = image_size[0].astype(jnp.float32)
    iw = image_size[1].astype(jnp.float32)
    scale = jnp.minimum(jnp.float32(H), jnp.float32(W)) / jnp.minimum(ih, iw)

    r = jnp.round(boxes * scale).astype(jnp.int32)
    rsw, rsh, rew, reh = r[:, 0], r[:, 1], r[:, 2], r[:, 3]
    roi_w = jnp.maximum(rew - rsw + 1, 1)[:, None]
    roi_h = jnp.maximum(reh - rsh + 1, 1)[:, None]
    bins = jnp.arange(P)[None, :]
    hs = jnp.clip((bins * roi_h) // P + rsh[:, None], 0, H)
    he = jnp.clip(-((-(bins + 1) * roi_h) // P) + rsh[:, None], 0, H)
    ws = jnp.clip((bins * roi_w) // P + rsw[:, None], 0, W)
    we = jnp.clip(-((-(bins + 1) * roi_w) // P) + rsw[:, None], 0, W)
    pad2 = jnp.zeros((N, LANES - 2 * P), jnp.int32)
    vh = jnp.concatenate([hs, he, pad2], axis=1).astype(jnp.int32)
    vw = jnp.concatenate([ws, we, pad2], axis=1).astype(jnp.int32)
    meta = jnp.stack([vh, vw], axis=1)

    n_blk = 16
    bpb = -(-N // (n_blk * 4)) * 4
    meta = jnp.pad(meta, ((0, n_blk * bpb - N), (0, 0), (0, 0)))
    meta = meta.reshape(n_blk, bpb * 2 * LANES)

    fmap2 = (
        feature[0]
        .transpose(1, 2, 0)
        .reshape(H * W, 2, half_c)
        .transpose(1, 0, 2)
        .reshape(2, H * W * half_c)
    )

    out = _roi_pool_sc(fmap2, meta, N, C, H, W, n_blk, bpb)
    out = out.reshape(N, 2, P * P, half_c).transpose(0, 1, 3, 2)
    return out.reshape(N, C, P, P)
